# baseline (device time: 684238 ns/iter reference)
import jax
import jax.numpy as jnp
from jax import lax
from jax.experimental import pallas as pl
from jax.experimental.pallas import tpu as pltpu

N_DEV = 8
N_EXP_TOTAL = 32
E_LOC = 4
CAP = 204


def kernel(x, router_W, route_idx, expert_W):
    del router_W
    n_tok, d_model = x.shape
    e_loc, _, d_ff = expert_W.shape
    routes = route_idx.astype(jnp.int32)

    def body(x_ref, r_ref, ew_ref, out_ref,
             hist_all, w_comm, h_send, h_recv, w_send, w_recv, credit):
        my = lax.axis_index("i")
        left = lax.rem(my + N_DEV - 1, N_DEV)
        right = lax.rem(my + 1, N_DEV)

        barrier = pltpu.get_barrier_semaphore()
        for nbr in (left, right):
            pl.semaphore_signal(barrier, inc=1, device_id=(nbr,),
                                device_id_type=pl.DeviceIdType.MESH)
        pl.semaphore_wait(barrier, 2)

        rcol = r_ref[...]
        e_iota = lax.broadcasted_iota(jnp.int32, (1, N_EXP_TOTAL), 1)
        one_hot = (rcol == e_iota).astype(jnp.float32)

        hist_all[0:1, :] = jnp.sum(one_hot, axis=0, keepdims=True)

        for hop in range(N_DEV - 1):
            rdma = pltpu.make_async_remote_copy(
                src_ref=hist_all.at[pl.ds(hop, 1)],
                dst_ref=hist_all.at[pl.ds(hop + 1, 1)],
                send_sem=h_send.at[hop],
                recv_sem=h_recv.at[hop],
                device_id=(right,),
                device_id_type=pl.DeviceIdType.MESH,
            )
            rdma.start()
            rdma.wait()

        s_iota = lax.broadcasted_iota(jnp.int32, (N_DEV, 1), 0)
        svalid = ((s_iota >= 1) & (s_iota <= my)).astype(jnp.float32)
        base = jnp.sum(hist_all[...] * svalid, axis=0, keepdims=True)

        row_i = lax.broadcasted_iota(jnp.int32, (n_tok, n_tok), 0)
        col_i = lax.broadcasted_iota(jnp.int32, (n_tok, n_tok), 1)
        l_strict = (row_i > col_i).astype(jnp.float32)
        lrank = jnp.dot(l_strict, one_hot,
                        preferred_element_type=jnp.float32)
        kept = jnp.where(lrank + base < CAP, one_hot, 0.0)
        keep_col = jnp.sum(kept, axis=1, keepdims=True)

        xv = x_ref[...]
        out_ref[...] = jnp.zeros((n_tok, d_ff), jnp.float32)

        for h in range(N_DEV):
            o = lax.rem(my + (N_DEV - h), N_DEV)
            if h < N_DEV - 1:
                if h >= 2:
                    pl.semaphore_wait(credit, 1)
                rdma = pltpu.make_async_remote_copy(
                    src_ref=(ew_ref if h == 0 else w_comm.at[h % 2]),
                    dst_ref=w_comm.at[(h + 1) % 2],
                    send_sem=w_send.at[h],
                    recv_sem=w_recv.at[h],
                    device_id=(right,),
                    device_id_type=pl.DeviceIdType.MESH,
                )
                rdma.start()
            for j in range(E_LOC):
                e = o * E_LOC + j
                mask = keep_col * (rcol == e).astype(jnp.float32)
                xm = xv * mask
                wj = ew_ref[j] if h == 0 else w_comm[h % 2, j]
                out_ref[...] += jnp.dot(xm, wj,
                                        preferred_element_type=jnp.float32)
            if h < N_DEV - 1:
                rdma.wait()
                if 1 <= h <= N_DEV - 3:
                    pl.semaphore_signal(credit, inc=1, device_id=(left,),
                                        device_id_type=pl.DeviceIdType.MESH)

    params_cls = getattr(pltpu, "CompilerParams", None) or getattr(
        pltpu, "TPUCompilerParams"
    )
    return pl.pallas_call(
        body,
        out_shape=jax.ShapeDtypeStruct((n_tok, d_ff), jnp.float32),
        in_specs=[
            pl.BlockSpec(memory_space=pltpu.VMEM),
            pl.BlockSpec(memory_space=pltpu.VMEM),
            pl.BlockSpec(memory_space=pltpu.VMEM),
        ],
        out_specs=pl.BlockSpec(memory_space=pltpu.VMEM),
        scratch_shapes=[
            pltpu.VMEM((N_DEV, N_EXP_TOTAL), jnp.float32),
            pltpu.VMEM((2, e_loc, d_model, d_ff), jnp.float32),
            pltpu.SemaphoreType.DMA((N_DEV - 1,)),
            pltpu.SemaphoreType.DMA((N_DEV - 1,)),
            pltpu.SemaphoreType.DMA((N_DEV - 1,)),
            pltpu.SemaphoreType.DMA((N_DEV - 1,)),
            pltpu.SemaphoreType.REGULAR,
        ],
        compiler_params=params_cls(collective_id=0),
    )(x, routes, expert_W)


# device time: 404477 ns/iter; 1.6917x vs baseline; 1.6917x over previous
import jax
import jax.numpy as jnp
from jax import lax
from jax.experimental import pallas as pl
from jax.experimental.pallas import tpu as pltpu

N_DEV = 8
N_EXP_TOTAL = 32
E_LOC = 4
CAP = 204
R_HOPS = 4
L_HOPS = 3


def kernel(x, router_W, route_idx, expert_W):
    del router_W
    n_tok, d_model = x.shape
    e_loc, _, d_ff = expert_W.shape
    routes = route_idx.astype(jnp.int32)

    def body(x_ref, r_ref, ew_ref, out_ref,
             hist_all, rbuf, lbuf,
             h_send, h_recv, r_send, r_recv, l_send, l_recv,
             credit_r, credit_l):
        my = lax.axis_index("i")
        left = lax.rem(my + N_DEV - 1, N_DEV)
        right = lax.rem(my + 1, N_DEV)

        barrier = pltpu.get_barrier_semaphore()
        for nbr in (left, right):
            pl.semaphore_signal(barrier, inc=1, device_id=(nbr,),
                                device_id_type=pl.DeviceIdType.MESH)
        pl.semaphore_wait(barrier, 2)

        rcol = r_ref[...]
        e_iota = lax.broadcasted_iota(jnp.int32, (1, N_EXP_TOTAL), 1)
        one_hot = (rcol == e_iota).astype(jnp.float32)

        hist_all[0:1, :] = jnp.sum(one_hot, axis=0, keepdims=True)

        for hop in range(N_DEV - 1):
            rdma = pltpu.make_async_remote_copy(
                src_ref=hist_all.at[pl.ds(hop, 1)],
                dst_ref=hist_all.at[pl.ds(hop + 1, 1)],
                send_sem=h_send.at[hop],
                recv_sem=h_recv.at[hop],
                device_id=(right,),
                device_id_type=pl.DeviceIdType.MESH,
            )
            rdma.start()
            rdma.wait()

        s_iota = lax.broadcasted_iota(jnp.int32, (N_DEV, 1), 0)
        svalid = ((s_iota >= 1) & (s_iota <= my)).astype(jnp.float32)
        base = jnp.sum(hist_all[...] * svalid, axis=0, keepdims=True)

        csum = one_hot
        shift = 1
        while shift < n_tok:
            csum = csum + jnp.concatenate(
                [jnp.zeros((shift, N_EXP_TOTAL), jnp.float32),
                 csum[:-shift, :]], axis=0)
            shift *= 2
        lrank = csum - one_hot
        kept = jnp.where(lrank + base < CAP, one_hot, 0.0)
        keep_col = jnp.sum(kept, axis=1, keepdims=True)

        out_ref[...] = jnp.zeros((n_tok, d_ff), jnp.float32)

        def accum(origin, wref_getter):
            for j in range(E_LOC):
                e = origin * E_LOC + j
                mask = keep_col * (rcol == e).astype(jnp.float32)
                xm = x_ref[...] * mask
                out_ref[...] += jnp.dot(xm, wref_getter(j),
                                        preferred_element_type=jnp.float32)

        for h in range(R_HOPS + 1):
            rdma_r = rdma_l = None
            if h < R_HOPS:
                if h >= 2:
                    pl.semaphore_wait(credit_r, 1)
                rdma_r = pltpu.make_async_remote_copy(
                    src_ref=(ew_ref if h == 0 else rbuf.at[h % 2]),
                    dst_ref=rbuf.at[(h + 1) % 2],
                    send_sem=r_send.at[h],
                    recv_sem=r_recv.at[h],
                    device_id=(right,),
                    device_id_type=pl.DeviceIdType.MESH,
                )
                rdma_r.start()
            if h < L_HOPS:
                if h >= 2:
                    pl.semaphore_wait(credit_l, 1)
                rdma_l = pltpu.make_async_remote_copy(
                    src_ref=(ew_ref if h == 0 else lbuf.at[h % 2]),
                    dst_ref=lbuf.at[(h + 1) % 2],
                    send_sem=l_send.at[h],
                    recv_sem=l_recv.at[h],
                    device_id=(left,),
                    device_id_type=pl.DeviceIdType.MESH,
                )
                rdma_l.start()

            if h == 0:
                accum(my, lambda j: ew_ref[j])
            else:
                o_r = lax.rem(my + (N_DEV - h), N_DEV)
                accum(o_r, lambda j: rbuf[h % 2, j])
                if h <= L_HOPS:
                    o_l = lax.rem(my + h, N_DEV)
                    accum(o_l, lambda j: lbuf[h % 2, j])

            if rdma_r is not None:
                rdma_r.wait()
                if 1 <= h <= R_HOPS - 2:
                    pl.semaphore_signal(credit_r, inc=1, device_id=(left,),
                                        device_id_type=pl.DeviceIdType.MESH)
            if rdma_l is not None:
                rdma_l.wait()
                if 1 <= h <= L_HOPS - 2:
                    pl.semaphore_signal(credit_l, inc=1, device_id=(right,),
                                        device_id_type=pl.DeviceIdType.MESH)

    params_cls = getattr(pltpu, "CompilerParams", None) or getattr(
        pltpu, "TPUCompilerParams"
    )
    return pl.pallas_call(
        body,
        out_shape=jax.ShapeDtypeStruct((n_tok, d_ff), jnp.float32),
        in_specs=[
            pl.BlockSpec(memory_space=pltpu.VMEM),
            pl.BlockSpec(memory_space=pltpu.VMEM),
            pl.BlockSpec(memory_space=pltpu.VMEM),
        ],
        out_specs=pl.BlockSpec(memory_space=pltpu.VMEM),
        scratch_shapes=[
            pltpu.VMEM((N_DEV, N_EXP_TOTAL), jnp.float32),
            pltpu.VMEM((2, e_loc, d_model, d_ff), jnp.float32),
            pltpu.VMEM((2, e_loc, d_model, d_ff), jnp.float32),
            pltpu.SemaphoreType.DMA((N_DEV - 1,)),
            pltpu.SemaphoreType.DMA((N_DEV - 1,)),
            pltpu.SemaphoreType.DMA((R_HOPS,)),
            pltpu.SemaphoreType.DMA((R_HOPS,)),
            pltpu.SemaphoreType.DMA((L_HOPS,)),
            pltpu.SemaphoreType.DMA((L_HOPS,)),
            pltpu.SemaphoreType.REGULAR,
            pltpu.SemaphoreType.REGULAR,
        ],
        compiler_params=params_cls(
            collective_id=0, vmem_limit_bytes=100 * 1024 * 1024
        ),
    )(x, routes, expert_W)


# device time: 216426 ns/iter; 3.1615x vs baseline; 1.8689x over previous
import jax
import jax.numpy as jnp
from jax import lax
from jax.experimental import pallas as pl
from jax.experimental.pallas import tpu as pltpu

N_DEV = 8
N_EXP_TOTAL = 32
E_LOC = 4
CAP = 204
R_HOPS = 4
L_HOPS = 3


def kernel(x, router_W, route_idx, expert_W):
    del router_W
    n_tok, d_model = x.shape
    e_loc, _, d_ff = expert_W.shape
    routes = route_idx.astype(jnp.int32)

    def body(x_ref, r_ref, ew_ref, out_ref,
             hist_all, ew_bf, rbuf, lbuf,
             h_send, h_recv, r_send, r_recv, l_send, l_recv,
             credit_r, credit_l):
        my = lax.axis_index("i")
        left = lax.rem(my + N_DEV - 1, N_DEV)
        right = lax.rem(my + 1, N_DEV)

        barrier = pltpu.get_barrier_semaphore()
        for p in range(N_DEV):
            @pl.when(my != p)
            def _():
                pl.semaphore_signal(barrier, inc=1, device_id=(p,),
                                    device_id_type=pl.DeviceIdType.MESH)
        pl.semaphore_wait(barrier, N_DEV - 1)

        rcol = r_ref[...]
        e_iota = lax.broadcasted_iota(jnp.int32, (1, N_EXP_TOTAL), 1)
        one_hot = (rcol == e_iota).astype(jnp.float32)

        ew_bf[...] = ew_ref[...].astype(jnp.bfloat16)

        hist_all[pl.ds(my, 1), :] = jnp.sum(one_hot, axis=0, keepdims=True)

        for p in range(N_DEV):
            @pl.when(my != p)
            def _():
                rdma = pltpu.make_async_remote_copy(
                    src_ref=hist_all.at[pl.ds(my, 1)],
                    dst_ref=hist_all.at[pl.ds(my, 1)],
                    send_sem=h_send.at[p],
                    recv_sem=h_recv.at[my],
                    device_id=(p,),
                    device_id_type=pl.DeviceIdType.MESH,
                )
                rdma.start()
        for p in range(N_DEV):
            @pl.when(my != p)
            def _():
                desc = pltpu.make_async_remote_copy(
                    src_ref=hist_all.at[pl.ds(p, 1)],
                    dst_ref=hist_all.at[pl.ds(p, 1)],
                    send_sem=h_send.at[p],
                    recv_sem=h_recv.at[p],
                    device_id=(p,),
                    device_id_type=pl.DeviceIdType.MESH,
                )
                desc.wait_send()
                desc.wait_recv()

        s_iota = lax.broadcasted_iota(jnp.int32, (N_DEV, 1), 0)
        svalid = (s_iota < my).astype(jnp.float32)
        base = jnp.sum(hist_all[...] * svalid, axis=0, keepdims=True)

        csum = one_hot
        shift = 1
        while shift < n_tok:
            csum = csum + jnp.concatenate(
                [jnp.zeros((shift, N_EXP_TOTAL), jnp.float32),
                 csum[:-shift, :]], axis=0)
            shift *= 2
        lrank = csum - one_hot
        kept = jnp.where(lrank + base < CAP, one_hot, 0.0)
        keep_col = jnp.sum(kept, axis=1, keepdims=True)

        out_ref[...] = jnp.zeros((n_tok, d_ff), jnp.float32)

        def accum(origin, wref_getter):
            for j in range(E_LOC):
                e = origin * E_LOC + j
                mask = keep_col * (rcol == e).astype(jnp.float32)
                xm = (x_ref[...] * mask).astype(jnp.bfloat16)
                out_ref[...] += jnp.dot(xm, wref_getter(j),
                                        preferred_element_type=jnp.float32)

        for h in range(R_HOPS + 1):
            rdma_r = rdma_l = None
            if h < R_HOPS:
                if h >= 2:
                    pl.semaphore_wait(credit_r, 1)
                rdma_r = pltpu.make_async_remote_copy(
                    src_ref=(ew_bf if h == 0 else rbuf.at[h % 2]),
                    dst_ref=rbuf.at[(h + 1) % 2],
                    send_sem=r_send.at[h],
                    recv_sem=r_recv.at[h],
                    device_id=(right,),
                    device_id_type=pl.DeviceIdType.MESH,
                )
                rdma_r.start()
            if h < L_HOPS:
                if h >= 2:
                    pl.semaphore_wait(credit_l, 1)
                rdma_l = pltpu.make_async_remote_copy(
                    src_ref=(ew_bf if h == 0 else lbuf.at[h % 2]),
                    dst_ref=lbuf.at[(h + 1) % 2],
                    send_sem=l_send.at[h],
                    recv_sem=l_recv.at[h],
                    device_id=(left,),
                    device_id_type=pl.DeviceIdType.MESH,
                )
                rdma_l.start()

            if h == 0:
                accum(my, lambda j: ew_bf[j])
            else:
                o_r = lax.rem(my + (N_DEV - h), N_DEV)
                accum(o_r, lambda j: rbuf[h % 2, j])
                if h <= L_HOPS:
                    o_l = lax.rem(my + h, N_DEV)
                    accum(o_l, lambda j: lbuf[h % 2, j])

            if rdma_r is not None:
                rdma_r.wait()
                if 1 <= h <= R_HOPS - 2:
                    pl.semaphore_signal(credit_r, inc=1, device_id=(left,),
                                        device_id_type=pl.DeviceIdType.MESH)
            if rdma_l is not None:
                rdma_l.wait()
                if 1 <= h <= L_HOPS - 2:
                    pl.semaphore_signal(credit_l, inc=1, device_id=(right,),
                                        device_id_type=pl.DeviceIdType.MESH)

    params_cls = getattr(pltpu, "CompilerParams", None) or getattr(
        pltpu, "TPUCompilerParams"
    )
    return pl.pallas_call(
        body,
        out_shape=jax.ShapeDtypeStruct((n_tok, d_ff), jnp.float32),
        in_specs=[
            pl.BlockSpec(memory_space=pltpu.VMEM),
            pl.BlockSpec(memory_space=pltpu.VMEM),
            pl.BlockSpec(memory_space=pltpu.VMEM),
        ],
        out_specs=pl.BlockSpec(memory_space=pltpu.VMEM),
        scratch_shapes=[
            pltpu.VMEM((N_DEV, N_EXP_TOTAL), jnp.float32),
            pltpu.VMEM((e_loc, d_model, d_ff), jnp.bfloat16),
            pltpu.VMEM((2, e_loc, d_model, d_ff), jnp.bfloat16),
            pltpu.VMEM((2, e_loc, d_model, d_ff), jnp.bfloat16),
            pltpu.SemaphoreType.DMA((N_DEV,)),
            pltpu.SemaphoreType.DMA((N_DEV,)),
            pltpu.SemaphoreType.DMA((R_HOPS,)),
            pltpu.SemaphoreType.DMA((R_HOPS,)),
            pltpu.SemaphoreType.DMA((L_HOPS,)),
            pltpu.SemaphoreType.DMA((L_HOPS,)),
            pltpu.SemaphoreType.REGULAR,
            pltpu.SemaphoreType.REGULAR,
        ],
        compiler_params=params_cls(
            collective_id=0, vmem_limit_bytes=100 * 1024 * 1024
        ),
    )(x, routes, expert_W)


# device time: 196629 ns/iter; 3.4798x vs baseline; 1.1007x over previous
import jax
import jax.numpy as jnp
from jax import lax
from jax.experimental import pallas as pl
from jax.experimental.pallas import tpu as pltpu

N_DEV = 8
N_EXP_TOTAL = 32
E_LOC = 4
CAP = 204
HOPS = 4


def kernel(x, router_W, route_idx, expert_W):
    del router_W
    n_tok, d_model = x.shape
    e_loc, _, d_ff = expert_W.shape
    half = e_loc // 2
    routes = route_idx.astype(jnp.int32)

    def body(x_ref, r_ref, ew_ref, out_ref,
             hist_all, ew_bf, rbuf, lbuf,
             h_send, h_recv, r_send, r_recv, l_send, l_recv,
             credit_r, credit_l):
        my = lax.axis_index("i")
        left = lax.rem(my + N_DEV - 1, N_DEV)
        right = lax.rem(my + 1, N_DEV)

        barrier = pltpu.get_barrier_semaphore()
        for p in range(N_DEV):
            @pl.when(my != p)
            def _():
                pl.semaphore_signal(barrier, inc=1, device_id=(p,),
                                    device_id_type=pl.DeviceIdType.MESH)
        pl.semaphore_wait(barrier, N_DEV - 1)

        rcol = r_ref[...]
        e_iota = lax.broadcasted_iota(jnp.int32, (1, N_EXP_TOTAL), 1)
        one_hot = (rcol == e_iota).astype(jnp.float32)

        ew_bf[...] = ew_ref[...].astype(jnp.bfloat16)

        hist_all[pl.ds(my, 1), :] = jnp.sum(one_hot, axis=0, keepdims=True)

        def accum(origin, js, wref_getter):
            for j in js:
                e = origin * E_LOC + j
                mask = keep_col * (rcol == e).astype(jnp.float32)
                xm = (x_ref[...] * mask).astype(jnp.bfloat16)
                out_ref[...] += jnp.dot(xm, wref_getter(j),
                                        preferred_element_type=jnp.float32)

        for h in range(HOPS + 1):
            rdma_r = rdma_l = None
            if h < HOPS:
                if h >= 2:
                    pl.semaphore_wait(credit_r, 1)
                rdma_r = pltpu.make_async_remote_copy(
                    src_ref=(ew_bf if h == 0 else
                             rbuf.at[h % 2] if h < HOPS - 1 else
                             rbuf.at[h % 2, pl.ds(0, half)]),
                    dst_ref=(rbuf.at[(h + 1) % 2] if h < HOPS - 1 else
                             rbuf.at[(h + 1) % 2, pl.ds(0, half)]),
                    send_sem=r_send.at[h],
                    recv_sem=r_recv.at[h],
                    device_id=(right,),
                    device_id_type=pl.DeviceIdType.MESH,
                )
                rdma_r.start()
                if h >= 2:
                    pl.semaphore_wait(credit_l, 1)
                rdma_l = pltpu.make_async_remote_copy(
                    src_ref=(ew_bf if h == 0 else
                             lbuf.at[h % 2] if h < HOPS - 1 else
                             lbuf.at[h % 2, pl.ds(half, half)]),
                    dst_ref=(lbuf.at[(h + 1) % 2] if h < HOPS - 1 else
                             lbuf.at[(h + 1) % 2, pl.ds(half, half)]),
                    send_sem=l_send.at[h],
                    recv_sem=l_recv.at[h],
                    device_id=(left,),
                    device_id_type=pl.DeviceIdType.MESH,
                )
                rdma_l.start()

            if h == 0:
                for p in range(N_DEV):
                    @pl.when(my != p)
                    def _():
                        rdma = pltpu.make_async_remote_copy(
                            src_ref=hist_all.at[pl.ds(my, 1)],
                            dst_ref=hist_all.at[pl.ds(my, 1)],
                            send_sem=h_send.at[p],
                            recv_sem=h_recv.at[my],
                            device_id=(p,),
                            device_id_type=pl.DeviceIdType.MESH,
                        )
                        rdma.start()
                for p in range(N_DEV):
                    @pl.when(my != p)
                    def _():
                        desc = pltpu.make_async_remote_copy(
                            src_ref=hist_all.at[pl.ds(p, 1)],
                            dst_ref=hist_all.at[pl.ds(p, 1)],
                            send_sem=h_send.at[p],
                            recv_sem=h_recv.at[p],
                            device_id=(p,),
                            device_id_type=pl.DeviceIdType.MESH,
                        )
                        desc.wait_send()
                        desc.wait_recv()

                s_iota = lax.broadcasted_iota(jnp.int32, (N_DEV, 1), 0)
                svalid = (s_iota < my).astype(jnp.float32)
                base = jnp.sum(hist_all[...] * svalid,
                               axis=0, keepdims=True)
                csum = one_hot
                shift = 1
                while shift < n_tok:
                    csum = csum + jnp.concatenate(
                        [jnp.zeros((shift, N_EXP_TOTAL), jnp.float32),
                         csum[:-shift, :]], axis=0)
                    shift *= 2
                lrank = csum - one_hot
                kept = jnp.where(lrank + base < CAP, one_hot, 0.0)
                keep_col = jnp.sum(kept, axis=1, keepdims=True)

                out_ref[...] = jnp.zeros((n_tok, d_ff), jnp.float32)
                accum(my, range(E_LOC), lambda j: ew_bf[j])
            elif h < HOPS:
                o_r = lax.rem(my + (N_DEV - h), N_DEV)
                accum(o_r, range(E_LOC), lambda j: rbuf[h % 2, j])
                o_l = lax.rem(my + h, N_DEV)
                accum(o_l, range(E_LOC), lambda j: lbuf[h % 2, j])
            else:
                o = lax.rem(my + HOPS, N_DEV)
                accum(o, range(half), lambda j: rbuf[0, j])
                accum(o, range(half, E_LOC), lambda j: lbuf[0, j])

            if rdma_r is not None:
                rdma_r.wait()
                if 1 <= h <= HOPS - 2:
                    pl.semaphore_signal(credit_r, inc=1, device_id=(left,),
                                        device_id_type=pl.DeviceIdType.MESH)
            if rdma_l is not None:
                rdma_l.wait()
                if 1 <= h <= HOPS - 2:
                    pl.semaphore_signal(credit_l, inc=1, device_id=(right,),
                                        device_id_type=pl.DeviceIdType.MESH)

    params_cls = getattr(pltpu, "CompilerParams", None) or getattr(
        pltpu, "TPUCompilerParams"
    )
    return pl.pallas_call(
        body,
        out_shape=jax.ShapeDtypeStruct((n_tok, d_ff), jnp.float32),
        in_specs=[
            pl.BlockSpec(memory_space=pltpu.VMEM),
            pl.BlockSpec(memory_space=pltpu.VMEM),
            pl.BlockSpec(memory_space=pltpu.VMEM),
        ],
        out_specs=pl.BlockSpec(memory_space=pltpu.VMEM),
        scratch_shapes=[
            pltpu.VMEM((N_DEV, N_EXP_TOTAL), jnp.float32),
            pltpu.VMEM((e_loc, d_model, d_ff), jnp.bfloat16),
            pltpu.VMEM((2, e_loc, d_model, d_ff), jnp.bfloat16),
            pltpu.VMEM((2, e_loc, d_model, d_ff), jnp.bfloat16),
            pltpu.SemaphoreType.DMA((N_DEV,)),
            pltpu.SemaphoreType.DMA((N_DEV,)),
            pltpu.SemaphoreType.DMA((HOPS,)),
            pltpu.SemaphoreType.DMA((HOPS,)),
            pltpu.SemaphoreType.DMA((HOPS,)),
            pltpu.SemaphoreType.DMA((HOPS,)),
            pltpu.SemaphoreType.REGULAR,
            pltpu.SemaphoreType.REGULAR,
        ],
        compiler_params=params_cls(
            collective_id=0, vmem_limit_bytes=100 * 1024 * 1024
        ),
    )(x, routes, expert_W)


# device time: 196614 ns/iter; 3.4801x vs baseline; 1.0001x over previous
import jax
import jax.numpy as jnp
from jax import lax
from jax.experimental import pallas as pl
from jax.experimental.pallas import tpu as pltpu

N_DEV = 8
N_EXP_TOTAL = 32
E_LOC = 4
CAP = 204
HOPS = 4


def kernel(x, router_W, route_idx, expert_W):
    del router_W
    n_tok, d_model = x.shape
    e_loc, _, d_ff = expert_W.shape
    half = e_loc // 2
    routes = route_idx.astype(jnp.int32)

    def body(x_ref, r_ref, ew_ref, out_ref,
             hist_all, ew_bf, rbuf, lbuf,
             h_send, h_recv, r_send, r_recv, l_send, l_recv,
             credit_r, credit_l):
        my = lax.axis_index("i")
        left = lax.rem(my + N_DEV - 1, N_DEV)
        right = lax.rem(my + 1, N_DEV)

        barrier = pltpu.get_barrier_semaphore()
        for p in range(N_DEV):
            @pl.when(my != p)
            def _():
                pl.semaphore_signal(barrier, inc=1, device_id=(p,),
                                    device_id_type=pl.DeviceIdType.MESH)
        pl.semaphore_wait(barrier, N_DEV - 1)

        rcol = r_ref[...]
        e_iota = lax.broadcasted_iota(jnp.int32, (1, N_EXP_TOTAL), 1)
        one_hot = (rcol == e_iota).astype(jnp.float32)

        ew_bf[...] = ew_ref[...].astype(jnp.bfloat16)

        hist_all[pl.ds(my, 1), :] = jnp.sum(one_hot, axis=0, keepdims=True)

        def accum(origin, js, wval):
            parts = []
            for j in js:
                e = origin * E_LOC + j
                mb = (keep_col * (rcol == e).astype(jnp.float32)
                      ).astype(jnp.bfloat16)
                parts.append(xb * mb)
            xcat = jnp.concatenate(parts, axis=1) if len(parts) > 1 else parts[0]
            wcat = wval.reshape(len(js) * d_model, d_ff)
            out_ref[...] += jnp.dot(xcat, wcat,
                                    preferred_element_type=jnp.float32)

        for h in range(HOPS + 1):
            rdma_r = rdma_l = None
            if h < HOPS:
                if h >= 2:
                    pl.semaphore_wait(credit_r, 1)
                rdma_r = pltpu.make_async_remote_copy(
                    src_ref=(ew_bf if h == 0 else
                             rbuf.at[h % 2] if h < HOPS - 1 else
                             rbuf.at[h % 2, pl.ds(0, half)]),
                    dst_ref=(rbuf.at[(h + 1) % 2] if h < HOPS - 1 else
                             rbuf.at[(h + 1) % 2, pl.ds(0, half)]),
                    send_sem=r_send.at[h],
                    recv_sem=r_recv.at[h],
                    device_id=(right,),
                    device_id_type=pl.DeviceIdType.MESH,
                )
                rdma_r.start()
                if h >= 2:
                    pl.semaphore_wait(credit_l, 1)
                rdma_l = pltpu.make_async_remote_copy(
                    src_ref=(ew_bf if h == 0 else
                             lbuf.at[h % 2] if h < HOPS - 1 else
                             lbuf.at[h % 2, pl.ds(half, half)]),
                    dst_ref=(lbuf.at[(h + 1) % 2] if h < HOPS - 1 else
                             lbuf.at[(h + 1) % 2, pl.ds(half, half)]),
                    send_sem=l_send.at[h],
                    recv_sem=l_recv.at[h],
                    device_id=(left,),
                    device_id_type=pl.DeviceIdType.MESH,
                )
                rdma_l.start()

            if h == 0:
                for p in range(N_DEV):
                    @pl.when(my != p)
                    def _():
                        rdma = pltpu.make_async_remote_copy(
                            src_ref=hist_all.at[pl.ds(my, 1)],
                            dst_ref=hist_all.at[pl.ds(my, 1)],
                            send_sem=h_send.at[p],
                            recv_sem=h_recv.at[my],
                            device_id=(p,),
                            device_id_type=pl.DeviceIdType.MESH,
                        )
                        rdma.start()
                for p in range(N_DEV):
                    @pl.when(my != p)
                    def _():
                        desc = pltpu.make_async_remote_copy(
                            src_ref=hist_all.at[pl.ds(p, 1)],
                            dst_ref=hist_all.at[pl.ds(p, 1)],
                            send_sem=h_send.at[p],
                            recv_sem=h_recv.at[p],
                            device_id=(p,),
                            device_id_type=pl.DeviceIdType.MESH,
                        )
                        desc.wait_send()
                        desc.wait_recv()

                s_iota = lax.broadcasted_iota(jnp.int32, (N_DEV, 1), 0)
                svalid = (s_iota < my).astype(jnp.float32)
                base = jnp.sum(hist_all[...] * svalid,
                               axis=0, keepdims=True)
                csum = one_hot
                shift = 1
                while shift < n_tok:
                    csum = csum + jnp.concatenate(
                        [jnp.zeros((shift, N_EXP_TOTAL), jnp.float32),
                         csum[:-shift, :]], axis=0)
                    shift *= 2
                lrank = csum - one_hot
                kept = jnp.where(lrank + base < CAP, one_hot, 0.0)
                keep_col = jnp.sum(kept, axis=1, keepdims=True)

                xb = x_ref[...].astype(jnp.bfloat16)
                out_ref[...] = jnp.zeros((n_tok, d_ff), jnp.float32)
                accum(my, range(E_LOC), ew_bf[...])
            elif h < HOPS:
                o_r = lax.rem(my + (N_DEV - h), N_DEV)
                accum(o_r, range(E_LOC), rbuf[h % 2])
                o_l = lax.rem(my + h, N_DEV)
                accum(o_l, range(E_LOC), lbuf[h % 2])
            else:
                o = lax.rem(my + HOPS, N_DEV)
                accum(o, range(half), rbuf[0, 0:half])
                accum(o, range(half, E_LOC), lbuf[0, half:E_LOC])

            if rdma_r is not None:
                rdma_r.wait()
                if 1 <= h <= HOPS - 2:
                    pl.semaphore_signal(credit_r, inc=1, device_id=(left,),
                                        device_id_type=pl.DeviceIdType.MESH)
            if rdma_l is not None:
                rdma_l.wait()
                if 1 <= h <= HOPS - 2:
                    pl.semaphore_signal(credit_l, inc=1, device_id=(right,),
                                        device_id_type=pl.DeviceIdType.MESH)

    params_cls = getattr(pltpu, "CompilerParams", None) or getattr(
        pltpu, "TPUCompilerParams"
    )
    return pl.pallas_call(
        body,
        out_shape=jax.ShapeDtypeStruct((n_tok, d_ff), jnp.float32),
        in_specs=[
            pl.BlockSpec(memory_space=pltpu.VMEM),
            pl.BlockSpec(memory_space=pltpu.VMEM),
            pl.BlockSpec(memory_space=pltpu.VMEM),
        ],
        out_specs=pl.BlockSpec(memory_space=pltpu.VMEM),
        scratch_shapes=[
            pltpu.VMEM((N_DEV, N_EXP_TOTAL), jnp.float32),
            pltpu.VMEM((e_loc, d_model, d_ff), jnp.bfloat16),
            pltpu.VMEM((2, e_loc, d_model, d_ff), jnp.bfloat16),
            pltpu.VMEM((2, e_loc, d_model, d_ff), jnp.bfloat16),
            pltpu.SemaphoreType.DMA((N_DEV,)),
            pltpu.SemaphoreType.DMA((N_DEV,)),
            pltpu.SemaphoreType.DMA((HOPS,)),
            pltpu.SemaphoreType.DMA((HOPS,)),
            pltpu.SemaphoreType.DMA((HOPS,)),
            pltpu.SemaphoreType.DMA((HOPS,)),
            pltpu.SemaphoreType.REGULAR,
            pltpu.SemaphoreType.REGULAR,
        ],
        compiler_params=params_cls(
            collective_id=0, vmem_limit_bytes=100 * 1024 * 1024
        ),
    )(x, routes, expert_W)
